# Initial kernel scaffold; baseline (speedup 1.0000x reference)
import jax
import jax.numpy as jnp
from jax.experimental import pallas as pl


def kernel(drone_feats, boxes, drone_mask, params):
    raise NotImplementedError("write your pallas kernel here")


# speedup vs baseline: 3.4163x; 3.4163x over previous
import jax
import jax.numpy as jnp
from jax import lax
from jax.experimental import pallas as pl

_B, _T, _M = 2, 8, 128
_IN, _GNN, _H, _C = 256, 256, 8, 32
_TEMP, _OUT, _NL = 256, 256, 2
_NHEAD, _DH = 8, 32
_FF = 2 * _TEMP
_DIST_TH = 0.3
_BT = _B * _T


def _ln2d(x, g, b):
    mu = jnp.mean(x, axis=-1, keepdims=True)
    d = x - mu
    v = jnp.mean(d * d, axis=-1, keepdims=True)
    return d * lax.rsqrt(v + 1e-5) * g + b


def _gat_layer(x, relx, rely, dist, eye, adjf, md, mx, my,
               wT, ws, wdT, weT, b):
    AS = lax.dot_general(ws, x, (((1,), (1,)), ((), ())),
                         preferred_element_type=jnp.float32)
    AD = jnp.dot(x, wdT, preferred_element_type=jnp.float32)
    outs = []
    for h in range(_H):
        w0 = weT[0:1, h:h + 1]
        w1 = weT[1:2, h:h + 1]
        w2 = weT[2:3, h:h + 1]
        ae = dist * w0 + relx * w1 + rely * w2
        aed = md * w0 + mx * w1 + my * w2
        ae = jnp.where(eye, aed, ae)
        lg = AS[h:h + 1, :] + AD[:, h:h + 1] + ae
        lg = jnp.where(lg > 0, lg, 0.2 * lg)
        lg = jnp.where(adjf, lg, -1e9)
        m = jnp.max(lg, axis=1, keepdims=True)
        e = jnp.exp(lg - m)
        al = e / jnp.sum(e, axis=1, keepdims=True)
        xp_h = jnp.dot(x, wT[:, h * _C:(h + 1) * _C],
                       preferred_element_type=jnp.float32)
        outs.append(jnp.dot(al, xp_h, preferred_element_type=jnp.float32))
    return jnp.concatenate(outs, axis=1) + b


def _frame_kernel(feats_ref, pxc_ref, pxr_ref, pyc_ref, pyr_ref,
                  mc_ref, mr_ref, winT_ref, bin_ref,
                  wT0_ref, ws0_ref, wdT0_ref, weT0_ref, b0_ref, g0_ref, e0_ref,
                  wT1_ref, ws1_ref, wdT1_ref, weT1_ref, b1_ref, g1_ref, e1_ref,
                  out_ref):
    feats = feats_ref[0]
    pxc = pxc_ref[0]
    pxr = pxr_ref[0]
    pyc = pyc_ref[0]
    pyr = pyr_ref[0]
    mc = mc_ref[0]
    mr = mr_ref[0]

    relx = pxc - pxr
    rely = pyc - pyr
    sq = relx * relx + rely * rely
    ii = lax.broadcasted_iota(jnp.int32, (_M, _M), 0)
    jj = lax.broadcasted_iota(jnp.int32, (_M, _M), 1)
    eye = ii == jj
    dist = jnp.sqrt(sq + eye.astype(jnp.float32) + 1e-12)
    vb = (mc > 0.5) & (mr > 0.5)
    adj = (dist < _DIST_TH) & (~eye) & vb
    adjf = adj | eye
    adjm = adj.astype(jnp.float32)
    inv = 1.0 / jnp.maximum(jnp.sum(adjm, keepdims=True), 1.0)
    md = jnp.sum(dist * adjm, keepdims=True) * inv
    mx = jnp.sum(relx * adjm, keepdims=True) * inv
    my = jnp.sum(rely * adjm, keepdims=True) * inv

    x = jnp.dot(feats, winT_ref[...],
                preferred_element_type=jnp.float32) + bin_ref[...]

    layer_refs = ((wT0_ref, ws0_ref, wdT0_ref, weT0_ref, b0_ref, g0_ref, e0_ref),
                  (wT1_ref, ws1_ref, wdT1_ref, weT1_ref, b1_ref, g1_ref, e1_ref))
    for (wT, ws, wdT, weT, b, g, e) in layer_refs:
        res = x
        o = _gat_layer(x, relx, rely, dist, eye, adjf, md, mx, my,
                       wT[...], ws[...], wdT[...], weT[...], b[...])
        x = _ln2d(o + res, g[...], e[...])
        x = jnp.maximum(x, 0.0)
    out_ref[0] = jnp.mean(x, axis=0, keepdims=True)


def _temporal_kernel(ff_ref, wtT_ref, bt_ref, pos_ref,
                     inwT0, inb0, owT0, ob0, l1g0, l1b0, l2g0, l2b0,
                     f1wT0, f1b0, f2wT0, f2b0,
                     inwT1, inb1, owT1, ob1, l1g1, l1b1, l2g1, l2b1,
                     f1wT1, f1b1, f2wT1, f2b1,
                     poolw_ref, poolb_ref, outwT_ref, outb_ref, og_ref, ob2_ref,
                     out_ref):
    x = jnp.dot(ff_ref[...], wtT_ref[...],
                preferred_element_type=jnp.float32) + bt_ref[...] + pos_ref[...]
    bi = lax.broadcasted_iota(jnp.int32, (_BT, _BT), 0) // _T
    bj = lax.broadcasted_iota(jnp.int32, (_BT, _BT), 1) // _T
    blk = bi == bj
    scale = 1.0 / (_DH ** 0.5)

    layers = ((inwT0, inb0, owT0, ob0, l1g0, l1b0, l2g0, l2b0,
               f1wT0, f1b0, f2wT0, f2b0),
              (inwT1, inb1, owT1, ob1, l1g1, l1b1, l2g1, l2b1,
               f1wT1, f1b1, f2wT1, f2b1))
    for (inwT, inb, owT, ob, l1g, l1b, l2g, l2b,
         f1wT, f1b, f2wT, f2b) in layers:
        hn = _ln2d(x, l1g[...], l1b[...])
        qkv = jnp.dot(hn, inwT[...],
                      preferred_element_type=jnp.float32) + inb[...]
        heads = []
        for h in range(_NHEAD):
            qh = qkv[:, h * _DH:(h + 1) * _DH]
            kh = qkv[:, _TEMP + h * _DH:_TEMP + (h + 1) * _DH]
            vh = qkv[:, 2 * _TEMP + h * _DH:2 * _TEMP + (h + 1) * _DH]
            s = lax.dot_general(qh, kh, (((1,), (1,)), ((), ())),
                                preferred_element_type=jnp.float32) * scale
            s = jnp.where(blk, s, -1e9)
            m = jnp.max(s, axis=1, keepdims=True)
            es = jnp.exp(s - m)
            a = es / jnp.sum(es, axis=1, keepdims=True)
            heads.append(jnp.dot(a, vh, preferred_element_type=jnp.float32))
        o = jnp.concatenate(heads, axis=1)
        x = x + jnp.dot(o, owT[...],
                        preferred_element_type=jnp.float32) + ob[...]
        hn = _ln2d(x, l2g[...], l2b[...])
        f = jnp.maximum(jnp.dot(hn, f1wT[...],
                                preferred_element_type=jnp.float32) + f1b[...],
                        0.0)
        x = x + jnp.dot(f, f2wT[...],
                        preferred_element_type=jnp.float32) + f2b[...]

    sc = lax.dot_general(poolw_ref[...], x, (((1,), (1,)), ((), ())),
                         preferred_element_type=jnp.float32) + poolb_ref[...]
    scb = jnp.broadcast_to(sc, (_B, _BT))
    pi = lax.broadcasted_iota(jnp.int32, (_B, _BT), 0)
    pj = lax.broadcasted_iota(jnp.int32, (_B, _BT), 1) // _T
    pp = jnp.where(pi == pj, scb, -1e9)
    pm = jnp.max(pp, axis=1, keepdims=True)
    pe = jnp.exp(pp - pm)
    P = pe / jnp.sum(pe, axis=1, keepdims=True)
    pooled = jnp.dot(P, x, preferred_element_type=jnp.float32)
    y = _ln2d(jnp.dot(pooled, outwT_ref[...],
                      preferred_element_type=jnp.float32) + outb_ref[...],
              og_ref[...], ob2_ref[...])
    out_ref[...] = jnp.maximum(y, 0.0)


def kernel(drone_feats, boxes, drone_mask, params):
    p = params
    f32 = jnp.float32
    feats = drone_feats.reshape(_BT, _M, _IN)
    pos = boxes.reshape(_BT, _M, 5)[:, :, 1:3]
    px = pos[:, :, 0]
    py = pos[:, :, 1]
    pxc = px[:, :, None]
    pxr = px[:, None, :]
    pyc = py[:, :, None]
    pyr = py[:, None, :]
    mk = drone_mask.reshape(_BT, _M)
    mc = mk[:, :, None]
    mr = mk[:, None, :]

    winT = p['W_in'].T
    bin_ = p['b_in'].reshape(1, _GNN)
    layer_args = []
    for l in range(_NL):
        W = p['gat%d_W' % l]
        Wh = W.reshape(_H, _C, _GNN)
        ws = (Wh * p['gat%d_as' % l][:, :, None]).sum(1)
        wdT = (Wh * p['gat%d_ad' % l][:, :, None]).sum(1).T
        weT = (p['gat%d_We' % l].reshape(_H, _C, 3)
               * p['gat%d_ae' % l][:, :, None]).sum(1).T
        layer_args += [W.T, ws, wdT, weT,
                       p['gat%d_b' % l].reshape(1, -1),
                       p['gat%d_lng' % l].reshape(1, -1),
                       p['gat%d_lnb' % l].reshape(1, -1)]

    row = lambda last: pl.BlockSpec((1, 1, last), lambda i: (i, 0, 0))
    col = pl.BlockSpec((1, _M, 1), lambda i: (i, 0, 0))
    full2 = lambda a: pl.BlockSpec(a.shape, lambda i: (0, 0))
    in_specs = [pl.BlockSpec((1, _M, _IN), lambda i: (i, 0, 0)),
                col, row(_M), col, row(_M), col, row(_M)]
    in_specs += [full2(winT), full2(bin_)]
    in_specs += [full2(a) for a in layer_args]

    ff = pl.pallas_call(
        _frame_kernel,
        grid=(_BT,),
        in_specs=in_specs,
        out_specs=pl.BlockSpec((1, 1, _GNN), lambda i: (i, 0, 0)),
        out_shape=jax.ShapeDtypeStruct((_BT, 1, _GNN), f32),
    )(feats, pxc, pxr, pyc, pyr, mc, mr, winT, bin_, *layer_args)
    ff = ff.reshape(_BT, _GNN)

    pos2 = jnp.tile(p['pos_emb'][0, :_T, :], (_B, 1))
    targs = [ff, p['W_temp'].T, p['b_temp'].reshape(1, -1), pos2]
    for l in range(2):
        targs += [p['t%d_inw' % l].T, p['t%d_inb' % l].reshape(1, -1),
                  p['t%d_ow' % l].T, p['t%d_ob' % l].reshape(1, -1),
                  p['t%d_ln1g' % l].reshape(1, -1),
                  p['t%d_ln1b' % l].reshape(1, -1),
                  p['t%d_ln2g' % l].reshape(1, -1),
                  p['t%d_ln2b' % l].reshape(1, -1),
                  p['t%d_f1w' % l].T, p['t%d_f1b' % l].reshape(1, -1),
                  p['t%d_f2w' % l].T, p['t%d_f2b' % l].reshape(1, -1)]
    targs += [p['pool_w'], p['pool_b'].reshape(1, 1),
              p['out_w'].T, p['out_b'].reshape(1, -1),
              p['olng'].reshape(1, -1), p['olnb'].reshape(1, -1)]

    return pl.pallas_call(
        _temporal_kernel,
        out_shape=jax.ShapeDtypeStruct((_B, _OUT), f32),
    )(*targs)


# speedup vs baseline: 3.8202x; 1.1182x over previous
import jax
import jax.numpy as jnp
from jax import lax
from jax.experimental import pallas as pl

_B, _T, _M = 2, 8, 128
_IN, _GNN, _H, _C = 256, 256, 8, 32
_TEMP, _OUT, _NL = 256, 256, 2
_NHEAD, _DH = 8, 32
_FF = 2 * _TEMP
_DIST_TH = 0.3
_BT = _B * _T
_FPB = 4


def _ln2d(x, g, b):
    mu = jnp.mean(x, axis=-1, keepdims=True)
    d = x - mu
    v = jnp.mean(d * d, axis=-1, keepdims=True)
    return d * lax.rsqrt(v + 1e-5) * g + b


def _gat_layer(x, relx, rely, dist, eye, adjf, md, mx, my,
               wT, ws, wdT, weT, b):
    AS = lax.dot_general(ws, x, (((1,), (1,)), ((), ())),
                         preferred_element_type=jnp.float32)
    AD = jnp.dot(x, wdT, preferred_element_type=jnp.float32)
    outs = []
    for h in range(_H):
        w0 = weT[0:1, h:h + 1]
        w1 = weT[1:2, h:h + 1]
        w2 = weT[2:3, h:h + 1]
        ae = dist * w0 + relx * w1 + rely * w2
        aed = md * w0 + mx * w1 + my * w2
        ae = jnp.where(eye, aed, ae)
        lg = AS[h:h + 1, :] + AD[:, h:h + 1] + ae
        lg = jnp.where(lg > 0, lg, 0.2 * lg)
        lg = jnp.where(adjf, lg, -1e9)
        m = jnp.max(lg, axis=1, keepdims=True)
        e = jnp.exp(lg - m)
        al = e / jnp.sum(e, axis=1, keepdims=True)
        xp_h = jnp.dot(x, wT[:, h * _C:(h + 1) * _C],
                       preferred_element_type=jnp.float32)
        outs.append(jnp.dot(al, xp_h, preferred_element_type=jnp.float32))
    return jnp.concatenate(outs, axis=1) + b


def _one_frame(feats, pxc, pxr, pyc, pyr, mc, mr, winT, bin_, layer_ws):
    relx = pxc - pxr
    rely = pyc - pyr
    sq = relx * relx + rely * rely
    ii = lax.broadcasted_iota(jnp.int32, (_M, _M), 0)
    jj = lax.broadcasted_iota(jnp.int32, (_M, _M), 1)
    eye = ii == jj
    dist = jnp.sqrt(sq + eye.astype(jnp.float32) + 1e-12)
    vb = (mc > 0.5) & (mr > 0.5)
    adj = (dist < _DIST_TH) & (~eye) & vb
    adjf = adj | eye
    adjm = adj.astype(jnp.float32)
    inv = 1.0 / jnp.maximum(jnp.sum(adjm, keepdims=True), 1.0)
    md = jnp.sum(dist * adjm, keepdims=True) * inv
    mx = jnp.sum(relx * adjm, keepdims=True) * inv
    my = jnp.sum(rely * adjm, keepdims=True) * inv

    x = jnp.dot(feats, winT, preferred_element_type=jnp.float32) + bin_
    for (wT, ws, wdT, weT, b, g, e) in layer_ws:
        res = x
        o = _gat_layer(x, relx, rely, dist, eye, adjf, md, mx, my,
                       wT, ws, wdT, weT, b)
        x = _ln2d(o + res, g, e)
        x = jnp.maximum(x, 0.0)
    return jnp.mean(x, axis=0, keepdims=True)


def _frame_kernel(feats_ref, pxc_ref, pxr_ref, pyc_ref, pyr_ref,
                  mc_ref, mr_ref, winT_ref, bin_ref,
                  wT0_ref, ws0_ref, wdT0_ref, weT0_ref, b0_ref, g0_ref, e0_ref,
                  wT1_ref, ws1_ref, wdT1_ref, weT1_ref, b1_ref, g1_ref, e1_ref,
                  out_ref):
    winT = winT_ref[...]
    bin_ = bin_ref[...]
    layer_ws = ((wT0_ref[...], ws0_ref[...], wdT0_ref[...], weT0_ref[...],
                 b0_ref[...], g0_ref[...], e0_ref[...]),
                (wT1_ref[...], ws1_ref[...], wdT1_ref[...], weT1_ref[...],
                 b1_ref[...], g1_ref[...], e1_ref[...]))
    for f in range(_FPB):
        out_ref[f] = _one_frame(feats_ref[f], pxc_ref[f], pxr_ref[f],
                                pyc_ref[f], pyr_ref[f], mc_ref[f], mr_ref[f],
                                winT, bin_, layer_ws)


def _temporal_kernel(ff_ref, wtT_ref, bt_ref, pos_ref,
                     inwT0, inb0, owT0, ob0, l1g0, l1b0, l2g0, l2b0,
                     f1wT0, f1b0, f2wT0, f2b0,
                     inwT1, inb1, owT1, ob1, l1g1, l1b1, l2g1, l2b1,
                     f1wT1, f1b1, f2wT1, f2b1,
                     poolw_ref, poolb_ref, outwT_ref, outb_ref, og_ref, ob2_ref,
                     out_ref):
    x = jnp.dot(ff_ref[...], wtT_ref[...],
                preferred_element_type=jnp.float32) + bt_ref[...] + pos_ref[...]
    bi = lax.broadcasted_iota(jnp.int32, (_BT, _BT), 0) // _T
    bj = lax.broadcasted_iota(jnp.int32, (_BT, _BT), 1) // _T
    blk = bi == bj
    scale = 1.0 / (_DH ** 0.5)

    layers = ((inwT0, inb0, owT0, ob0, l1g0, l1b0, l2g0, l2b0,
               f1wT0, f1b0, f2wT0, f2b0),
              (inwT1, inb1, owT1, ob1, l1g1, l1b1, l2g1, l2b1,
               f1wT1, f1b1, f2wT1, f2b1))
    for (inwT, inb, owT, ob, l1g, l1b, l2g, l2b,
         f1wT, f1b, f2wT, f2b) in layers:
        hn = _ln2d(x, l1g[...], l1b[...])
        qkv = jnp.dot(hn, inwT[...],
                      preferred_element_type=jnp.float32) + inb[...]
        heads = []
        for h in range(_NHEAD):
            qh = qkv[:, h * _DH:(h + 1) * _DH]
            kh = qkv[:, _TEMP + h * _DH:_TEMP + (h + 1) * _DH]
            vh = qkv[:, 2 * _TEMP + h * _DH:2 * _TEMP + (h + 1) * _DH]
            s = lax.dot_general(qh, kh, (((1,), (1,)), ((), ())),
                                preferred_element_type=jnp.float32) * scale
            s = jnp.where(blk, s, -1e9)
            m = jnp.max(s, axis=1, keepdims=True)
            es = jnp.exp(s - m)
            a = es / jnp.sum(es, axis=1, keepdims=True)
            heads.append(jnp.dot(a, vh, preferred_element_type=jnp.float32))
        o = jnp.concatenate(heads, axis=1)
        x = x + jnp.dot(o, owT[...],
                        preferred_element_type=jnp.float32) + ob[...]
        hn = _ln2d(x, l2g[...], l2b[...])
        f = jnp.maximum(jnp.dot(hn, f1wT[...],
                                preferred_element_type=jnp.float32) + f1b[...],
                        0.0)
        x = x + jnp.dot(f, f2wT[...],
                        preferred_element_type=jnp.float32) + f2b[...]

    sc = lax.dot_general(poolw_ref[...], x, (((1,), (1,)), ((), ())),
                         preferred_element_type=jnp.float32) + poolb_ref[...]
    scb = jnp.broadcast_to(sc, (_B, _BT))
    pi = lax.broadcasted_iota(jnp.int32, (_B, _BT), 0)
    pj = lax.broadcasted_iota(jnp.int32, (_B, _BT), 1) // _T
    pp = jnp.where(pi == pj, scb, -1e9)
    pm = jnp.max(pp, axis=1, keepdims=True)
    pe = jnp.exp(pp - pm)
    P = pe / jnp.sum(pe, axis=1, keepdims=True)
    pooled = jnp.dot(P, x, preferred_element_type=jnp.float32)
    y = _ln2d(jnp.dot(pooled, outwT_ref[...],
                      preferred_element_type=jnp.float32) + outb_ref[...],
              og_ref[...], ob2_ref[...])
    out_ref[...] = jnp.maximum(y, 0.0)


def kernel(drone_feats, boxes, drone_mask, params):
    p = params
    f32 = jnp.float32
    feats = drone_feats.reshape(_BT, _M, _IN)
    pos = boxes.reshape(_BT, _M, 5)[:, :, 1:3]
    px = pos[:, :, 0]
    py = pos[:, :, 1]
    pxc = px[:, :, None]
    pxr = px[:, None, :]
    pyc = py[:, :, None]
    pyr = py[:, None, :]
    mk = drone_mask.reshape(_BT, _M)
    mc = mk[:, :, None]
    mr = mk[:, None, :]

    winT = p['W_in'].T
    bin_ = p['b_in'].reshape(1, _GNN)
    layer_args = []
    for l in range(_NL):
        W = p['gat%d_W' % l]
        Wh = W.reshape(_H, _C, _GNN)
        ws = (Wh * p['gat%d_as' % l][:, :, None]).sum(1)
        wdT = (Wh * p['gat%d_ad' % l][:, :, None]).sum(1).T
        weT = (p['gat%d_We' % l].reshape(_H, _C, 3)
               * p['gat%d_ae' % l][:, :, None]).sum(1).T
        layer_args += [W.T, ws, wdT, weT,
                       p['gat%d_b' % l].reshape(1, -1),
                       p['gat%d_lng' % l].reshape(1, -1),
                       p['gat%d_lnb' % l].reshape(1, -1)]

    row = lambda last: pl.BlockSpec((_FPB, 1, last), lambda i: (i, 0, 0))
    col = pl.BlockSpec((_FPB, _M, 1), lambda i: (i, 0, 0))
    full2 = lambda a: pl.BlockSpec(a.shape, lambda i: (0, 0))
    in_specs = [pl.BlockSpec((_FPB, _M, _IN), lambda i: (i, 0, 0)),
                col, row(_M), col, row(_M), col, row(_M)]
    in_specs += [full2(winT), full2(bin_)]
    in_specs += [full2(a) for a in layer_args]

    ff = pl.pallas_call(
        _frame_kernel,
        grid=(_BT // _FPB,),
        in_specs=in_specs,
        out_specs=pl.BlockSpec((_FPB, 1, _GNN), lambda i: (i, 0, 0)),
        out_shape=jax.ShapeDtypeStruct((_BT, 1, _GNN), f32),
    )(feats, pxc, pxr, pyc, pyr, mc, mr, winT, bin_, *layer_args)
    ff = ff.reshape(_BT, _GNN)

    pos2 = jnp.tile(p['pos_emb'][0, :_T, :], (_B, 1))
    targs = [ff, p['W_temp'].T, p['b_temp'].reshape(1, -1), pos2]
    for l in range(2):
        targs += [p['t%d_inw' % l].T, p['t%d_inb' % l].reshape(1, -1),
                  p['t%d_ow' % l].T, p['t%d_ob' % l].reshape(1, -1),
                  p['t%d_ln1g' % l].reshape(1, -1),
                  p['t%d_ln1b' % l].reshape(1, -1),
                  p['t%d_ln2g' % l].reshape(1, -1),
                  p['t%d_ln2b' % l].reshape(1, -1),
                  p['t%d_f1w' % l].T, p['t%d_f1b' % l].reshape(1, -1),
                  p['t%d_f2w' % l].T, p['t%d_f2b' % l].reshape(1, -1)]
    targs += [p['pool_w'], p['pool_b'].reshape(1, 1),
              p['out_w'].T, p['out_b'].reshape(1, -1),
              p['olng'].reshape(1, -1), p['olnb'].reshape(1, -1)]

    return pl.pallas_call(
        _temporal_kernel,
        out_shape=jax.ShapeDtypeStruct((_B, _OUT), f32),
    )(*targs)
